# Initial kernel scaffold; baseline (speedup 1.0000x reference)
import jax
import jax.numpy as jnp
from jax.experimental import pallas as pl


def kernel(x, edge_index, W1, as1, ad1, b1, W2, as2, ad2, b2, Wfc, bfc):
    raise NotImplementedError("write your pallas kernel here")


# speedup vs baseline: 20.8236x; 20.8236x over previous
import jax
import jax.numpy as jnp
from jax import lax
from jax.experimental import pallas as pl
from jax.experimental.pallas import tpu as pltpu
from jax.experimental.pallas import tpu_sc as plsc

N = 100000
E_RAW = 1600000
E_TOT = E_RAW + N
W = 1024
WA = 256
NSC, NSUB = 2, 16
NWORK = NSC * NSUB
E_PAD = ((E_TOT + NWORK * W - 1) // (NWORK * W)) * (NWORK * W)
NW_A = E_PAD // (NWORK * WA)
NW_A2 = E_PAD // (NWORK * W)
NW_B = E_PAD // (NSUB * WA)
NPAD = 100352
STRIPE_A = NPAD // NSUB
CH = 50176
STRIPE_B = CH // NSUB
CH_REAL = 50000
ZRA = 392
ZRB = 196

_mesh = plsc.VectorSubcoreMesh(core_axis_name="c", subcore_axis_name="s")
_sc_params = pltpu.CompilerParams(use_tc_tiling_on_sc=False)

_f32 = jnp.float32


def _row(ref, i):
    return ref.at[pl.ds(i, 1), :][...]


def _phase_a(src, dst, atabs, atabd):
    @pl.kernel(
        out_type=[
            jax.ShapeDtypeStruct((E_PAD, 16), _f32),
            jax.ShapeDtypeStruct((NSC * NPAD, 16), _f32),
        ],
        mesh=_mesh,
        compiler_params=_sc_params,
        scratch_types=[
            pltpu.VMEM_SHARED((NPAD, 16), _f32),
            pltpu.VMEM((WA,), jnp.int32),
            pltpu.VMEM((WA,), jnp.int32),
            pltpu.VMEM((WA, 16), _f32),
            pltpu.VMEM((WA, 16), _f32),
            pltpu.VMEM((WA, 16), _f32),
            pltpu.VMEM((ZRA, 16), _f32),
        ],
    )
    def k(src_h, dst_h, ats_h, atd_h, eex_h, sp_h, sacc, srcb, dstb, asb,
          adb, exb, zb):
        c = lax.axis_index("c")
        s = lax.axis_index("s")

        @pl.loop(0, ZRA)
        def _(i):
            zb.at[pl.ds(i, 1), :][...] = jnp.zeros((1, 16), _f32)

        row0 = s * STRIPE_A

        @pl.loop(0, STRIPE_A // ZRA)
        def _(i):
            pltpu.sync_copy(zb, sacc.at[pl.ds(row0 + i * ZRA, ZRA)])

        plsc.subcore_barrier()

        wk = c * NSUB + s

        @pl.loop(0, NW_A)
        def _(j):
            base = (wk * NW_A + j) * WA
            pltpu.sync_copy(src_h.at[pl.ds(base, WA)], srcb)
            pltpu.sync_copy(dst_h.at[pl.ds(base, WA)], dstb)
            pltpu.sync_copy(ats_h.at[srcb], asb)
            pltpu.sync_copy(atd_h.at[dstb], adb)

            @pl.loop(0, WA)
            def _(i):
                ev = _row(asb, i) + _row(adb, i)
                ev = jnp.where(ev > 0, ev, 0.2 * ev)
                exb.at[pl.ds(i, 1), :][...] = jnp.exp(ev)

            pltpu.sync_copy(exb, eex_h.at[pl.ds(base, WA)])
            pltpu.sync_copy(exb, sacc.at[dstb], add=True)

        plsc.subcore_barrier()
        pltpu.sync_copy(sacc.at[pl.ds(row0, STRIPE_A)],
                        sp_h.at[pl.ds(c * NPAD + row0, STRIPE_A)])

    return k(src, dst, atabs, atabd)


def _phase_a2(dst, eex, sinv):
    @pl.kernel(
        out_type=jax.ShapeDtypeStruct((E_PAD, 16), _f32),
        mesh=_mesh,
        compiler_params=_sc_params,
        scratch_types=[
            pltpu.VMEM((WA,), jnp.int32),
            pltpu.VMEM((WA, 16), _f32),
            pltpu.VMEM((WA, 16), _f32),
            pltpu.VMEM((WA, 16), _f32),
        ],
    )
    def k(dst_h, eex_h, sinv_h, al_h, dstb, exr, svr, alr):
        c = lax.axis_index("c")
        s = lax.axis_index("s")
        wk = c * NSUB + s

        @pl.loop(0, NW_A)
        def _(j):
            base = (wk * NW_A + j) * WA
            pltpu.sync_copy(dst_h.at[pl.ds(base, WA)], dstb)
            pltpu.sync_copy(eex_h.at[pl.ds(base, WA)], exr)
            pltpu.sync_copy(sinv_h.at[dstb], svr)

            @pl.loop(0, WA)
            def _(i):
                alr.at[pl.ds(i, 1), :][...] = _row(exr, i) * _row(svr, i)

            pltpu.sync_copy(alr, al_h.at[pl.ds(base, WA)])

    return k(dst, eex, sinv)


def _phase_b(src, dst, alpha, htab, col):
    @pl.kernel(
        out_type=jax.ShapeDtypeStruct((NSC * CH, 32), _f32),
        mesh=_mesh,
        compiler_params=_sc_params,
        scratch_types=[
            pltpu.VMEM_SHARED((CH, 32), _f32),
            pltpu.VMEM((WA,), jnp.int32),
            pltpu.VMEM((WA,), jnp.int32),
            pltpu.VMEM((WA,), jnp.int32),
            pltpu.VMEM((WA, 32), _f32),
            pltpu.VMEM((WA, 16), _f32),
            pltpu.VMEM((ZRB, 32), _f32),
        ],
    )
    def k(src_h, dst_h, al_h, htab_h, out_h, acc, srcb, dstb, idxb, hr, ar,
          zb):
        c = lax.axis_index("c")
        s = lax.axis_index("s")

        @pl.loop(0, ZRB)
        def _(i):
            zb.at[pl.ds(i, 1), pl.ds(0, 16)][...] = jnp.zeros((1, 16), _f32)
            zb.at[pl.ds(i, 1), pl.ds(16, 16)][...] = jnp.zeros((1, 16), _f32)

        row0 = s * STRIPE_B

        @pl.loop(0, STRIPE_B // ZRB)
        def _(i):
            pltpu.sync_copy(zb, acc.at[pl.ds(row0 + i * ZRB, ZRB)])

        plsc.subcore_barrier()
        cbase = c * CH_REAL

        @pl.loop(0, NW_B)
        def _(j):
            base = (s * NW_B + j) * WA
            pltpu.sync_copy(src_h.at[pl.ds(base, WA)], srcb)
            pltpu.sync_copy(dst_h.at[pl.ds(base, WA)], dstb)
            pltpu.sync_copy(al_h.at[pl.ds(base, WA)], ar)
            pltpu.sync_copy(htab_h.at[srcb], hr)

            @pl.loop(0, WA // 16)
            def _(q):
                d16 = dstb[pl.ds(q * 16, 16)]
                loc = d16 - cbase
                ok = (loc >= 0) & (loc < CH_REAL)
                idxb[pl.ds(q * 16, 16)] = jnp.where(ok, loc, CH_REAL)

            @pl.loop(0, WA)
            def _(w):
                arow = ar.at[pl.ds(w, 1), :][...]
                av = jnp.broadcast_to(arow[:, col:col + 1], (1, 16))
                hr.at[pl.ds(w, 1), pl.ds(0, 16)][...] = (
                    hr.at[pl.ds(w, 1), pl.ds(0, 16)][...] * av)
                hr.at[pl.ds(w, 1), pl.ds(16, 16)][...] = (
                    hr.at[pl.ds(w, 1), pl.ds(16, 16)][...] * av)

            pltpu.sync_copy(hr, acc.at[idxb], add=True)

        plsc.subcore_barrier()
        pltpu.sync_copy(acc.at[pl.ds(row0, STRIPE_B)],
                        out_h.at[pl.ds(c * CH + row0, STRIPE_B)])

    return k(src, dst, alpha, htab)


def _unchunk(flat):
    return jnp.concatenate([flat[:CH_REAL], flat[CH : CH + CH_REAL]], axis=0)


BLK = 2000

def _selu(x):
    return 1.0507009873554805 * jnp.where(
        x > 0, x, 1.6732632423543772 * (jnp.exp(x) - 1.0))


def _tc1_body(x_ref, w1_ref, a1_ref, h0_ref, h1_ref, h2_ref, ats_ref,
              atd_ref):
    h = jnp.dot(x_ref[...], w1_ref[...], preferred_element_type=_f32)
    href = (h0_ref, h1_ref, h2_ref)
    cs, cd = [], []
    for hd in range(3):
        hh = h[:, hd * 32:(hd + 1) * 32]
        href[hd][...] = hh
        cs.append(jnp.dot(hh, a1_ref[0, hd * 32:(hd + 1) * 32].reshape(32, 1),
                          preferred_element_type=_f32))
        cd.append(jnp.dot(hh, a1_ref[1, hd * 32:(hd + 1) * 32].reshape(32, 1),
                          preferred_element_type=_f32))
    z = jnp.zeros((BLK, 13), _f32)
    ats_ref[...] = jnp.concatenate(cs + [z], axis=1)
    atd_ref[...] = jnp.concatenate(cd + [z], axis=1)


def _tc1(x, W1, a1):
    g = N // BLK
    return pl.pallas_call(
        _tc1_body,
        grid=(g,),
        in_specs=[
            pl.BlockSpec((BLK, 16), lambda i: (i, 0)),
            pl.BlockSpec((16, 96), lambda i: (0, 0)),
            pl.BlockSpec((2, 96), lambda i: (0, 0)),
        ],
        out_specs=[
            pl.BlockSpec((BLK, 32), lambda i: (i, 0)),
            pl.BlockSpec((BLK, 32), lambda i: (i, 0)),
            pl.BlockSpec((BLK, 32), lambda i: (i, 0)),
            pl.BlockSpec((BLK, 16), lambda i: (i, 0)),
            pl.BlockSpec((BLK, 16), lambda i: (i, 0)),
        ],
        out_shape=[jax.ShapeDtypeStruct((N, 32), _f32)] * 3
        + [jax.ShapeDtypeStruct((N, 16), _f32)] * 2,
    )(x, W1, a1)


def _sinv_body(p0_ref, p1_ref, o_ref):
    o_ref[...] = 1.0 / (p0_ref[...] + p1_ref[...] + 1e-16)


def _tc_sinv(sp):
    p0, p1 = sp[:NPAD], sp[NPAD:]
    g = NPAD // 2048
    return pl.pallas_call(
        _sinv_body,
        grid=(g,),
        in_specs=[pl.BlockSpec((2048, 16), lambda i: (i, 0))] * 2,
        out_specs=pl.BlockSpec((2048, 16), lambda i: (i, 0)),
        out_shape=jax.ShapeDtypeStruct((NPAD, 16), _f32),
    )(p0, p1)


def _tc2_body(o0_ref, o1_ref, o2_ref, b1_ref, w2_ref, a2_ref, h2_ref,
              ats_ref, atd_ref):
    g = jnp.concatenate([o0_ref[...], o1_ref[...], o2_ref[...]], axis=1)
    g = _selu(g + b1_ref[...])
    h2 = jnp.dot(g, w2_ref[...], preferred_element_type=_f32)
    h2_ref[...] = h2
    z = jnp.zeros((BLK, 15), _f32)
    ats_ref[...] = jnp.concatenate(
        [jnp.dot(h2, a2_ref[0].reshape(32, 1), preferred_element_type=_f32),
         z], axis=1)
    atd_ref[...] = jnp.concatenate(
        [jnp.dot(h2, a2_ref[1].reshape(32, 1), preferred_element_type=_f32),
         z], axis=1)


def _tc2(o0, o1, o2, b1, W2, a2):
    g = N // BLK
    return pl.pallas_call(
        _tc2_body,
        grid=(g,),
        in_specs=[
            pl.BlockSpec((BLK, 32), lambda i: (i, 0)),
            pl.BlockSpec((BLK, 32), lambda i: (i, 0)),
            pl.BlockSpec((BLK, 32), lambda i: (i, 0)),
            pl.BlockSpec((1, 96), lambda i: (0, 0)),
            pl.BlockSpec((96, 32), lambda i: (0, 0)),
            pl.BlockSpec((2, 32), lambda i: (0, 0)),
        ],
        out_specs=[
            pl.BlockSpec((BLK, 32), lambda i: (i, 0)),
            pl.BlockSpec((BLK, 16), lambda i: (i, 0)),
            pl.BlockSpec((BLK, 16), lambda i: (i, 0)),
        ],
        out_shape=[jax.ShapeDtypeStruct((N, 32), _f32),
                   jax.ShapeDtypeStruct((N, 16), _f32),
                   jax.ShapeDtypeStruct((N, 16), _f32)],
    )(o0, o1, o2, b1, W2, a2)


def _tc3_body(o_ref, b2_ref, wfc_ref, bfc_ref, out_ref):
    g = _selu(o_ref[...] + b2_ref[...])
    out_ref[...] = (jnp.dot(g, wfc_ref[...], preferred_element_type=_f32)
                    + bfc_ref[...])


def _tc3(o, b2, Wfc, bfc):
    g = N // BLK
    return pl.pallas_call(
        _tc3_body,
        grid=(g,),
        in_specs=[
            pl.BlockSpec((BLK, 32), lambda i: (i, 0)),
            pl.BlockSpec((1, 32), lambda i: (0, 0)),
            pl.BlockSpec((32, 32), lambda i: (0, 0)),
            pl.BlockSpec((1, 32), lambda i: (0, 0)),
        ],
        out_specs=pl.BlockSpec((BLK, 32), lambda i: (i, 0)),
        out_shape=jax.ShapeDtypeStruct((N, 32), _f32),
    )(o, b2, Wfc, bfc)


def _impl(x, edge_index, W1, as1, ad1, b1, W2, as2, ad2, b2, Wfc, bfc):
    loop = jnp.arange(N, dtype=edge_index.dtype)
    src = jnp.concatenate([edge_index[0], loop,
                           jnp.zeros((E_PAD - E_TOT,), jnp.int32)])
    dst = jnp.concatenate([edge_index[1], loop,
                           jnp.full((E_PAD - E_TOT,), N, jnp.int32)])

    a1 = jnp.stack([as1.reshape(96), ad1.reshape(96)])
    h0, h1, h2, ats, atd = _tc1(x.astype(_f32), W1, a1)
    pad = ((0, NPAD - N), (0, 0))
    eex, sp = _phase_a(src, dst, jnp.pad(ats, pad), jnp.pad(atd, pad))
    sinv = _tc_sinv(sp)
    alpha = _phase_a2(dst, eex, sinv)
    o0 = _unchunk(_phase_b(src, dst, alpha, h0, 0))
    o1 = _unchunk(_phase_b(src, dst, alpha, h1, 1))
    o2 = _unchunk(_phase_b(src, dst, alpha, h2, 2))

    a2 = jnp.stack([as2.reshape(32), ad2.reshape(32)])
    h2tab, ats2, atd2 = _tc2(o0, o1, o2, b1.reshape(1, 96), W2, a2)
    eex2, sp2 = _phase_a(src, dst, jnp.pad(ats2, pad), jnp.pad(atd2, pad))
    sinv2 = _tc_sinv(sp2)
    alpha2 = _phase_a2(dst, eex2, sinv2)
    og = _unchunk(_phase_b(src, dst, alpha2, h2tab, 0))

    return _tc3(og, b2.reshape(1, 32), Wfc, bfc.reshape(1, 32))


kernel = jax.jit(_impl)


# speedup vs baseline: 24.6101x; 1.1818x over previous
import jax
import jax.numpy as jnp
from jax import lax
from jax.experimental import pallas as pl
from jax.experimental.pallas import tpu as pltpu
from jax.experimental.pallas import tpu_sc as plsc

N = 100000
E_RAW = 1600000
E_TOT = E_RAW + N
W = 1024
WA = 256
NSC, NSUB = 2, 16
NWORK = NSC * NSUB
E_PAD = ((E_TOT + NWORK * W - 1) // (NWORK * W)) * (NWORK * W)
NW_A = E_PAD // (NWORK * WA)
NW_A2 = E_PAD // (NWORK * W)
NW_B = E_PAD // (NSUB * WA)
NPAD = 100352
STRIPE_A = NPAD // NSUB
CH = 50176
STRIPE_B = CH // NSUB
CH_REAL = 50000
ZRA = 392
ZRB = 196

_mesh = plsc.VectorSubcoreMesh(core_axis_name="c", subcore_axis_name="s")
_sc_params = pltpu.CompilerParams(use_tc_tiling_on_sc=False)

_f32 = jnp.float32


def _row(ref, i):
    return ref.at[pl.ds(i, 1), :][...]


def _phase_a(src, dst, atabs, atabd):
    @pl.kernel(
        out_type=[
            jax.ShapeDtypeStruct((E_PAD, 16), _f32),
            jax.ShapeDtypeStruct((NSC * NPAD, 16), _f32),
        ],
        mesh=_mesh,
        compiler_params=_sc_params,
        scratch_types=[
            pltpu.VMEM_SHARED((NPAD, 16), _f32),
            pltpu.VMEM((WA,), jnp.int32),
            pltpu.VMEM((WA,), jnp.int32),
            pltpu.VMEM((WA, 16), _f32),
            pltpu.VMEM((WA, 16), _f32),
            pltpu.VMEM((WA, 16), _f32),
            pltpu.VMEM((ZRA, 16), _f32),
            pltpu.SemaphoreType.DMA,
        ],
    )
    def k(src_h, dst_h, ats_h, atd_h, eex_h, sp_h, sacc, srcb, dstb, asb,
          adb, exb, zb, sem):
        c = lax.axis_index("c")
        s = lax.axis_index("s")

        @pl.loop(0, ZRA)
        def _(i):
            zb.at[pl.ds(i, 1), :][...] = jnp.zeros((1, 16), _f32)

        row0 = s * STRIPE_A

        @pl.loop(0, STRIPE_A // ZRA)
        def _(i):
            pltpu.sync_copy(zb, sacc.at[pl.ds(row0 + i * ZRA, ZRA)])

        plsc.subcore_barrier()

        wk = c * NSUB + s

        @pl.loop(0, NW_A)
        def _(j):
            base = (wk * NW_A + j) * WA
            c1 = pltpu.async_copy(src_h.at[pl.ds(base, WA)], srcb, sem)
            c2 = pltpu.async_copy(dst_h.at[pl.ds(base, WA)], dstb, sem)
            c1.wait()
            c2.wait()
            c3 = pltpu.async_copy(ats_h.at[srcb], asb, sem)
            c4 = pltpu.async_copy(atd_h.at[dstb], adb, sem)
            c3.wait()
            c4.wait()

            @pl.loop(0, WA)
            def _(i):
                ev = _row(asb, i) + _row(adb, i)
                ev = jnp.where(ev > 0, ev, 0.2 * ev)
                exb.at[pl.ds(i, 1), :][...] = jnp.exp(ev)

            pltpu.sync_copy(exb, eex_h.at[pl.ds(base, WA)])
            pltpu.sync_copy(exb, sacc.at[dstb], add=True)

        plsc.subcore_barrier()
        pltpu.sync_copy(sacc.at[pl.ds(row0, STRIPE_A)],
                        sp_h.at[pl.ds(c * NPAD + row0, STRIPE_A)])

    return k(src, dst, atabs, atabd)


def _phase_a2(dst, eex, sinv):
    @pl.kernel(
        out_type=jax.ShapeDtypeStruct((E_PAD, 16), _f32),
        mesh=_mesh,
        compiler_params=_sc_params,
        scratch_types=[
            pltpu.VMEM((WA,), jnp.int32),
            pltpu.VMEM((WA, 16), _f32),
            pltpu.VMEM((WA, 16), _f32),
            pltpu.VMEM((WA, 16), _f32),
        ],
    )
    def k(dst_h, eex_h, sinv_h, al_h, dstb, exr, svr, alr):
        c = lax.axis_index("c")
        s = lax.axis_index("s")
        wk = c * NSUB + s

        @pl.loop(0, NW_A)
        def _(j):
            base = (wk * NW_A + j) * WA
            pltpu.sync_copy(dst_h.at[pl.ds(base, WA)], dstb)
            pltpu.sync_copy(eex_h.at[pl.ds(base, WA)], exr)
            pltpu.sync_copy(sinv_h.at[dstb], svr)

            @pl.loop(0, WA)
            def _(i):
                alr.at[pl.ds(i, 1), :][...] = _row(exr, i) * _row(svr, i)

            pltpu.sync_copy(alr, al_h.at[pl.ds(base, WA)])

    return k(dst, eex, sinv)


def _phase_b(src, dst, alpha, htab, col):
    @pl.kernel(
        out_type=jax.ShapeDtypeStruct((NSC * CH, 32), _f32),
        mesh=_mesh,
        compiler_params=_sc_params,
        scratch_types=[
            pltpu.VMEM_SHARED((CH, 32), _f32),
            pltpu.VMEM((WA,), jnp.int32),
            pltpu.VMEM((WA,), jnp.int32),
            pltpu.VMEM((WA,), jnp.int32),
            pltpu.VMEM((WA, 32), _f32),
            pltpu.VMEM((WA, 16), _f32),
            pltpu.VMEM((ZRB, 32), _f32),
            pltpu.SemaphoreType.DMA,
        ],
    )
    def k(src_h, dst_h, al_h, htab_h, out_h, acc, srcb, dstb, idxb, hr, ar,
          zb, sem):
        c = lax.axis_index("c")
        s = lax.axis_index("s")

        @pl.loop(0, ZRB)
        def _(i):
            zb.at[pl.ds(i, 1), pl.ds(0, 16)][...] = jnp.zeros((1, 16), _f32)
            zb.at[pl.ds(i, 1), pl.ds(16, 16)][...] = jnp.zeros((1, 16), _f32)

        row0 = s * STRIPE_B

        @pl.loop(0, STRIPE_B // ZRB)
        def _(i):
            pltpu.sync_copy(zb, acc.at[pl.ds(row0 + i * ZRB, ZRB)])

        plsc.subcore_barrier()
        cbase = c * CH_REAL

        @pl.loop(0, NW_B)
        def _(j):
            base = (s * NW_B + j) * WA
            c1 = pltpu.async_copy(src_h.at[pl.ds(base, WA)], srcb, sem)
            c2 = pltpu.async_copy(dst_h.at[pl.ds(base, WA)], dstb, sem)
            c3 = pltpu.async_copy(al_h.at[pl.ds(base, WA)], ar, sem)
            c1.wait()
            c2.wait()
            c3.wait()
            pltpu.sync_copy(htab_h.at[srcb], hr)

            @pl.loop(0, WA // 16)
            def _(q):
                d16 = dstb[pl.ds(q * 16, 16)]
                loc = d16 - cbase
                ok = (loc >= 0) & (loc < CH_REAL)
                idxb[pl.ds(q * 16, 16)] = jnp.where(ok, loc, CH_REAL)

            @pl.loop(0, WA)
            def _(w):
                arow = ar.at[pl.ds(w, 1), :][...]
                av = jnp.broadcast_to(arow[:, col:col + 1], (1, 16))
                hr.at[pl.ds(w, 1), pl.ds(0, 16)][...] = (
                    hr.at[pl.ds(w, 1), pl.ds(0, 16)][...] * av)
                hr.at[pl.ds(w, 1), pl.ds(16, 16)][...] = (
                    hr.at[pl.ds(w, 1), pl.ds(16, 16)][...] * av)

            pltpu.sync_copy(hr, acc.at[idxb], add=True)

        plsc.subcore_barrier()
        pltpu.sync_copy(acc.at[pl.ds(row0, STRIPE_B)],
                        out_h.at[pl.ds(c * CH + row0, STRIPE_B)])

    return k(src, dst, alpha, htab)


def _unchunk(flat):
    return jnp.concatenate([flat[:CH_REAL], flat[CH : CH + CH_REAL]], axis=0)


BLK = 2000

def _selu(x):
    return 1.0507009873554805 * jnp.where(
        x > 0, x, 1.6732632423543772 * (jnp.exp(x) - 1.0))


def _tc1_body(x_ref, w1_ref, a1_ref, h0_ref, h1_ref, h2_ref, ats_ref,
              atd_ref):
    h = jnp.dot(x_ref[...], w1_ref[...], preferred_element_type=_f32)
    href = (h0_ref, h1_ref, h2_ref)
    cs, cd = [], []
    for hd in range(3):
        hh = h[:, hd * 32:(hd + 1) * 32]
        href[hd][...] = hh
        cs.append(jnp.dot(hh, a1_ref[0, hd * 32:(hd + 1) * 32].reshape(32, 1),
                          preferred_element_type=_f32))
        cd.append(jnp.dot(hh, a1_ref[1, hd * 32:(hd + 1) * 32].reshape(32, 1),
                          preferred_element_type=_f32))
    z = jnp.zeros((BLK, 13), _f32)
    ats_ref[...] = jnp.concatenate(cs + [z], axis=1)
    atd_ref[...] = jnp.concatenate(cd + [z], axis=1)


def _tc1(x, W1, a1):
    g = N // BLK
    return pl.pallas_call(
        _tc1_body,
        grid=(g,),
        in_specs=[
            pl.BlockSpec((BLK, 16), lambda i: (i, 0)),
            pl.BlockSpec((16, 96), lambda i: (0, 0)),
            pl.BlockSpec((2, 96), lambda i: (0, 0)),
        ],
        out_specs=[
            pl.BlockSpec((BLK, 32), lambda i: (i, 0)),
            pl.BlockSpec((BLK, 32), lambda i: (i, 0)),
            pl.BlockSpec((BLK, 32), lambda i: (i, 0)),
            pl.BlockSpec((BLK, 16), lambda i: (i, 0)),
            pl.BlockSpec((BLK, 16), lambda i: (i, 0)),
        ],
        out_shape=[jax.ShapeDtypeStruct((N, 32), _f32)] * 3
        + [jax.ShapeDtypeStruct((N, 16), _f32)] * 2,
    )(x, W1, a1)


def _sinv_body(p0_ref, p1_ref, o_ref):
    o_ref[...] = 1.0 / (p0_ref[...] + p1_ref[...] + 1e-16)


def _tc_sinv(sp):
    p0, p1 = sp[:NPAD], sp[NPAD:]
    g = NPAD // 2048
    return pl.pallas_call(
        _sinv_body,
        grid=(g,),
        in_specs=[pl.BlockSpec((2048, 16), lambda i: (i, 0))] * 2,
        out_specs=pl.BlockSpec((2048, 16), lambda i: (i, 0)),
        out_shape=jax.ShapeDtypeStruct((NPAD, 16), _f32),
    )(p0, p1)


def _tc2_body(o0_ref, o1_ref, o2_ref, b1_ref, w2_ref, a2_ref, h2_ref,
              ats_ref, atd_ref):
    g = jnp.concatenate([o0_ref[...], o1_ref[...], o2_ref[...]], axis=1)
    g = _selu(g + b1_ref[...])
    h2 = jnp.dot(g, w2_ref[...], preferred_element_type=_f32)
    h2_ref[...] = h2
    z = jnp.zeros((BLK, 15), _f32)
    ats_ref[...] = jnp.concatenate(
        [jnp.dot(h2, a2_ref[0].reshape(32, 1), preferred_element_type=_f32),
         z], axis=1)
    atd_ref[...] = jnp.concatenate(
        [jnp.dot(h2, a2_ref[1].reshape(32, 1), preferred_element_type=_f32),
         z], axis=1)


def _tc2(o0, o1, o2, b1, W2, a2):
    g = N // BLK
    return pl.pallas_call(
        _tc2_body,
        grid=(g,),
        in_specs=[
            pl.BlockSpec((BLK, 32), lambda i: (i, 0)),
            pl.BlockSpec((BLK, 32), lambda i: (i, 0)),
            pl.BlockSpec((BLK, 32), lambda i: (i, 0)),
            pl.BlockSpec((1, 96), lambda i: (0, 0)),
            pl.BlockSpec((96, 32), lambda i: (0, 0)),
            pl.BlockSpec((2, 32), lambda i: (0, 0)),
        ],
        out_specs=[
            pl.BlockSpec((BLK, 32), lambda i: (i, 0)),
            pl.BlockSpec((BLK, 16), lambda i: (i, 0)),
            pl.BlockSpec((BLK, 16), lambda i: (i, 0)),
        ],
        out_shape=[jax.ShapeDtypeStruct((N, 32), _f32),
                   jax.ShapeDtypeStruct((N, 16), _f32),
                   jax.ShapeDtypeStruct((N, 16), _f32)],
    )(o0, o1, o2, b1, W2, a2)


def _tc3_body(o_ref, b2_ref, wfc_ref, bfc_ref, out_ref):
    g = _selu(o_ref[...] + b2_ref[...])
    out_ref[...] = (jnp.dot(g, wfc_ref[...], preferred_element_type=_f32)
                    + bfc_ref[...])


def _tc3(o, b2, Wfc, bfc):
    g = N // BLK
    return pl.pallas_call(
        _tc3_body,
        grid=(g,),
        in_specs=[
            pl.BlockSpec((BLK, 32), lambda i: (i, 0)),
            pl.BlockSpec((1, 32), lambda i: (0, 0)),
            pl.BlockSpec((32, 32), lambda i: (0, 0)),
            pl.BlockSpec((1, 32), lambda i: (0, 0)),
        ],
        out_specs=pl.BlockSpec((BLK, 32), lambda i: (i, 0)),
        out_shape=jax.ShapeDtypeStruct((N, 32), _f32),
    )(o, b2, Wfc, bfc)


def _impl(x, edge_index, W1, as1, ad1, b1, W2, as2, ad2, b2, Wfc, bfc):
    loop = jnp.arange(N, dtype=edge_index.dtype)
    src = jnp.concatenate([edge_index[0], loop,
                           jnp.zeros((E_PAD - E_TOT,), jnp.int32)])
    dst = jnp.concatenate([edge_index[1], loop,
                           jnp.full((E_PAD - E_TOT,), N, jnp.int32)])

    a1 = jnp.stack([as1.reshape(96), ad1.reshape(96)])
    h0, h1, h2, ats, atd = _tc1(x.astype(_f32), W1, a1)
    pad = ((0, NPAD - N), (0, 0))
    eex, sp = _phase_a(src, dst, jnp.pad(ats, pad), jnp.pad(atd, pad))
    sinv = _tc_sinv(sp)
    alpha = _phase_a2(dst, eex, sinv)
    o0 = _unchunk(_phase_b(src, dst, alpha, h0, 0))
    o1 = _unchunk(_phase_b(src, dst, alpha, h1, 1))
    o2 = _unchunk(_phase_b(src, dst, alpha, h2, 2))

    a2 = jnp.stack([as2.reshape(32), ad2.reshape(32)])
    h2tab, ats2, atd2 = _tc2(o0, o1, o2, b1.reshape(1, 96), W2, a2)
    eex2, sp2 = _phase_a(src, dst, jnp.pad(ats2, pad), jnp.pad(atd2, pad))
    sinv2 = _tc_sinv(sp2)
    alpha2 = _phase_a2(dst, eex2, sinv2)
    og = _unchunk(_phase_b(src, dst, alpha2, h2tab, 0))

    return _tc3(og, b2.reshape(1, 32), Wfc, bfc.reshape(1, 32))


kernel = jax.jit(_impl)
